# Initial kernel scaffold; baseline (speedup 1.0000x reference)
import functools

import jax
import jax.numpy as jnp
from jax import lax
from jax.experimental import pallas as pl
from jax.experimental.pallas import tpu as pltpu
from jax.experimental.pallas import tpu_sc as plsc

NC = 2
NS = 16
CH = 128


@functools.partial(jax.jit, static_argnums=(4, 5, 6))
def _run(fh, src, dst, zeros, N, D, EP):
    DH = D // NC
    per_tile = EP // NS
    n_chunks = per_tile // CH
    racc = -(-(N + 1) // NS) * NS
    zrows = racc // NS
    orows = N // NS

    mesh = plsc.VectorSubcoreMesh(core_axis_name="c", subcore_axis_name="s")

    @functools.partial(
        pl.kernel,
        out_type=jax.ShapeDtypeStruct((N, D), jnp.float32),
        mesh=mesh,
        scratch_types=[
            pltpu.VMEM_SHARED((racc, DH), jnp.float32),
            pltpu.VMEM((CH,), jnp.int32),
            pltpu.VMEM((CH,), jnp.int32),
            pltpu.VMEM((CH, DH), jnp.float32),
            pltpu.SemaphoreType.DMA,
        ],
    )
    def k(fh_hbm, src_hbm, dst_hbm, zeros_hbm, out_hbm, acc, src_v, dst_v,
          rows_v, sem):
        c = lax.axis_index("c")
        s = lax.axis_index("s")
        pltpu.sync_copy(zeros_hbm, acc.at[pl.ds(s * zrows, zrows)])
        plsc.subcore_barrier()

        base = s * per_tile

        def step(i, carry):
            off = base + i * CH
            pltpu.sync_copy(src_hbm.at[pl.ds(off, CH)], src_v)
            pltpu.sync_copy(dst_hbm.at[pl.ds(off, CH)], dst_v)
            pltpu.async_copy(fh_hbm.at[c].at[src_v], rows_v, sem).wait()
            pltpu.sync_copy(rows_v, acc.at[dst_v], add=True)
            return carry

        lax.fori_loop(0, n_chunks, step, 0)
        plsc.subcore_barrier()
        pltpu.sync_copy(
            acc.at[pl.ds(s * orows, orows)],
            out_hbm.at[pl.ds(s * orows, orows), pl.ds(c * DH, DH)])

    return k(fh, src, dst, zeros)


def kernel(feature, edge_index):
    N, D = feature.shape
    E = edge_index.shape[1]
    per_tile = -(-(-(-E // NS)) // CH) * CH
    EP = per_tile * NS
    pad = EP - E
    src = jnp.concatenate(
        [edge_index[0].astype(jnp.int32), jnp.zeros((pad,), jnp.int32)])
    dst = jnp.concatenate(
        [edge_index[1].astype(jnp.int32), jnp.full((pad,), N, jnp.int32)])
    fh = feature.reshape(N, NC, D // NC).transpose(1, 0, 2)
    racc = -(-(N + 1) // NS) * NS
    zeros = jnp.zeros((racc // NS, D // NC), jnp.float32)
    return _run(fh, src, dst, zeros, N, D, EP)


# speedup vs baseline: 4.2899x; 4.2899x over previous
import functools

import jax
import jax.numpy as jnp
from jax import lax
from jax.experimental import pallas as pl
from jax.experimental.pallas import tpu as pltpu
from jax.experimental.pallas import tpu_sc as plsc

NC = 2
NS = 16
CH = 128


@functools.partial(jax.jit, static_argnums=(4, 5, 6))
def _run(feature, src, dst, zeros, N, D, EP):
    per_tile = EP // (NC * NS)
    n_chunks = per_tile // CH
    nup = -(-(N + 1) // (8 * NS)) * (8 * NS)
    zrows = nup // NS

    mesh = plsc.VectorSubcoreMesh(core_axis_name="c", subcore_axis_name="s")

    @functools.partial(
        pl.kernel,
        out_type=jax.ShapeDtypeStruct((NC, nup, D), jnp.float32),
        mesh=mesh,
        scratch_types=[
            pltpu.VMEM_SHARED((nup, D), jnp.float32),
            pltpu.VMEM((CH,), jnp.int32),
            pltpu.VMEM((CH,), jnp.int32),
            pltpu.VMEM((CH, D), jnp.float32),
            pltpu.SemaphoreType.DMA,
        ],
    )
    def k(feat_hbm, src_hbm, dst_hbm, zeros_hbm, part_hbm, acc, src_v, dst_v,
          rows_v, sem):
        c = lax.axis_index("c")
        s = lax.axis_index("s")
        pltpu.sync_copy(zeros_hbm, acc.at[pl.ds(s * zrows, zrows)])
        plsc.subcore_barrier()

        base = (s * NC + c) * per_tile

        def step(i, carry):
            off = base + i * CH
            pltpu.sync_copy(src_hbm.at[pl.ds(off, CH)], src_v)
            pltpu.sync_copy(dst_hbm.at[pl.ds(off, CH)], dst_v)
            pltpu.async_copy(feat_hbm.at[src_v], rows_v, sem).wait()
            pltpu.sync_copy(rows_v, acc.at[dst_v], add=True)
            return carry

        lax.fori_loop(0, n_chunks, step, 0)
        plsc.subcore_barrier()
        pltpu.sync_copy(acc.at[pl.ds(s * zrows, zrows)],
                        part_hbm.at[c].at[pl.ds(s * zrows, zrows)])

    part = k(feature, src, dst, zeros)

    rb = 1000
    grid = (N // rb,)

    def add_body(p_ref, o_ref):
        o_ref[...] = p_ref[0] + p_ref[1]

    return pl.pallas_call(
        add_body,
        grid=grid,
        in_specs=[pl.BlockSpec((NC, rb, D), lambda i: (0, i, 0))],
        out_specs=pl.BlockSpec((rb, D), lambda i: (i, 0)),
        out_shape=jax.ShapeDtypeStruct((N, D), jnp.float32),
    )(part)


def kernel(feature, edge_index):
    N, D = feature.shape
    E = edge_index.shape[1]
    nw = NC * NS
    per_tile = -(-(-(-E // nw)) // CH) * CH
    EP = per_tile * nw
    pad = EP - E
    src = jnp.concatenate(
        [edge_index[0].astype(jnp.int32), jnp.zeros((pad,), jnp.int32)])
    dst = jnp.concatenate(
        [edge_index[1].astype(jnp.int32), jnp.full((pad,), N, jnp.int32)])
    nup = -(-(N + 1) // (8 * NS)) * (8 * NS)
    zeros = jnp.zeros((nup // NS, D), jnp.float32)
    return _run(feature, src, dst, zeros, N, D, EP)
